# baseline (speedup 1.0000x reference)
import functools

import jax
import jax.numpy as jnp
from jax import lax
from jax.experimental import pallas as pl
from jax.experimental.pallas import tpu as pltpu
from jax.experimental.pallas import tpu_sc as plsc

VOCAB = 1000000
EMB = 64
NC = 2
NS = 16
NW = NC * NS
NPAIR = VOCAB // 2
NFULL = VOCAB // 128
TAIL_V0 = NFULL * 128
TAIL_W = 4

_MESH = plsc.VectorSubcoreMesh(
    core_axis_name="c", subcore_axis_name="s",
    num_cores=NC, num_subcores=NS,
)
_PARAMS = pltpu.CompilerParams(
    use_tc_tiling_on_sc=True, needs_layout_passes=False
)


def _wid():
    return lax.axis_index("s") * NC + lax.axis_index("c")


@functools.partial(
    pl.kernel,
    out_type=jax.ShapeDtypeStruct((NPAIR, 128), jnp.float32),
    mesh=_MESH,
    scratch_types=[
        pltpu.VMEM((2, EMB, 128), jnp.float32),
        pltpu.VMEM((2, EMB, 128), jnp.float32),
        pltpu.SemaphoreType.DMA((2,)),
        pltpu.SemaphoreType.DMA((2,)),
    ],
    compiler_params=_PARAMS,
)
def _phase_a(emb_t, tail2, tabp, inb, outb, isem, osem):
    w = _wid()
    iota = lax.iota(jnp.int32, 16)
    nblk = 244 + jnp.where(w < 4, 1, 0)

    def start_in(i):
        g = i * NW + w
        pltpu.async_copy(
            emb_t.at[:, pl.ds(g * 128, 128)], inb.at[i % 2], isem.at[i % 2]
        )

    def shuffle(src, dst, nrows):
        @plsc.parallel_loop(0, nrows, 1, unroll=16)
        def _row(r):
            for k in range(8):
                rows = iota + (16 * (k % 4))
                cols = jnp.full((16,), 2 * r + (k // 4), jnp.int32)
                dst[r, pl.ds(16 * k, 16)] = plsc.load_gather(src, [rows, cols])

    start_in(0)

    def step(i, _):
        buf = lax.rem(i, 2)
        g = i * NW + w

        @pl.when(i + 1 < nblk)
        def _():
            start_in(i + 1)

        @pl.when(g < NFULL)
        def _():
            pltpu.make_async_copy(
                emb_t.at[:, pl.ds(0, 128)], inb.at[buf], isem.at[buf]
            ).wait()
            @pl.when(i >= 2)
            def _():
                pltpu.make_async_copy(
                    outb.at[buf], tabp.at[pl.ds(0, EMB)], osem.at[buf]
                ).wait()
            shuffle(inb.at[buf], outb.at[buf], EMB)
            pltpu.async_copy(
                outb.at[buf], tabp.at[pl.ds(g * EMB, EMB)], osem.at[buf]
            )
        return ()

    lax.fori_loop(0, 245, step, (), unroll=False)
    for b in range(2):
        pltpu.make_async_copy(
            outb.at[b], tabp.at[pl.ds(0, EMB)], osem.at[b]
        ).wait()

    @pl.when(w == TAIL_W)
    def _():
        pltpu.sync_copy(tail2, outb.at[0, pl.ds(0, 32)])
        pltpu.sync_copy(outb.at[0, pl.ds(0, 32)],
                        tabp.at[pl.ds(NFULL * EMB, 32)])


@functools.partial(
    pl.kernel,
    out_type=jax.ShapeDtypeStruct((200, EMB, 4096), jnp.float32),
    mesh=_MESH,
    scratch_types=[
        pltpu.VMEM((2, 128), jnp.int32),
        pltpu.VMEM((2, 128), jnp.int32),
        pltpu.VMEM((2, 128), jnp.int32),
        pltpu.VMEM((2, 128, 128), jnp.float32),
        pltpu.VMEM((2, EMB, 128), jnp.float32),
        pltpu.SemaphoreType.DMA((2,)),
        pltpu.SemaphoreType.DMA((2,)),
        pltpu.SemaphoreType.DMA((2,)),
    ],
    compiler_params=_PARAMS,
)
def _phase_b(tok_t, tabp, outp, idxb, rb, hb, rowb, outt, isem, gsem, wsem):
    w = _wid()
    iota = lax.iota(jnp.int32, 16)
    nunits = 200

    def unit_tb(i):
        u = w * nunits + i
        return u // 32, lax.rem(u, 32)

    def start_idx(i):
        t, bb = unit_tb(i)
        pltpu.async_copy(
            tok_t.at[t, pl.ds(bb * 128, 128)], idxb.at[i % 2], isem.at[i % 2]
        )

    start_idx(0)

    def step(i, _):
        buf = lax.rem(i, 2)
        pbuf = lax.rem(i + 1, 2)

        @pl.when(i + 1 < nunits)
        def _():
            start_idx(i + 1)

        pltpu.make_async_copy(
            tok_t.at[0, pl.ds(0, 128)], idxb.at[buf], isem.at[buf]
        ).wait()
        for k in range(8):
            v = idxb[buf, pl.ds(16 * k, 16)]
            rb[buf, pl.ds(16 * k, 16)] = v >> 1
            hb[buf, pl.ds(16 * k, 16)] = (v & 1) << 6
        pltpu.async_copy(tabp.at[rb.at[buf]], rowb.at[buf], gsem.at[buf])

        @pl.when(i >= 1)
        def _():
            t, bb = unit_tb(i - 1)
            pltpu.make_async_copy(
                tabp.at[pl.ds(0, 128)], rowb.at[pbuf], gsem.at[pbuf]
            ).wait()

            @pl.when(i >= 3)
            def _():
                pltpu.make_async_copy(
                    outt.at[pbuf], outp.at[0, :, pl.ds(0, 128)], wsem.at[pbuf]
                ).wait()

            @plsc.parallel_loop(0, EMB, 1, unroll=16)
            def _erow(e):
                for k in range(8):
                    rows = iota + (16 * k)
                    cols = hb[pbuf, pl.ds(16 * k, 16)] + e
                    outt[pbuf, e, pl.ds(16 * k, 16)] = plsc.load_gather(
                        rowb.at[pbuf], [rows, cols]
                    )
            pltpu.async_copy(
                outt.at[pbuf], outp.at[t, :, pl.ds(bb * 128, 128)],
                wsem.at[pbuf],
            )
        return ()

    lax.fori_loop(0, nunits, step, (), unroll=False)

    last = lax.rem(nunits - 1, 2)
    t, bb = unit_tb(nunits - 1)
    pltpu.make_async_copy(
        tabp.at[pl.ds(0, 128)], rowb.at[last], gsem.at[last]
    ).wait()
    pltpu.make_async_copy(
        outt.at[last], outp.at[0, :, pl.ds(0, 128)], wsem.at[last]
    ).wait()

    @plsc.parallel_loop(0, EMB, 1, unroll=16)
    def _erow(e):
        for k in range(8):
            rows = iota + (16 * k)
            cols = hb[last, pl.ds(16 * k, 16)] + e
            outt[last, e, pl.ds(16 * k, 16)] = plsc.load_gather(
                rowb.at[last], [rows, cols]
            )
    pltpu.sync_copy(outt.at[last], outp.at[t, :, pl.ds(bb * 128, 128)])
    pltpu.make_async_copy(
        outt.at[1 - last], outp.at[0, :, pl.ds(0, 128)], wsem.at[1 - last]
    ).wait()


def kernel(token_ids, embeddings):
    tok_t = token_ids.T
    emb_t = embeddings.T
    tail2 = embeddings[TAIL_V0:].reshape(32, 128)
    tabp = _phase_a(emb_t, tail2)
    out_phys = _phase_b(tok_t, tabp)
    return jnp.transpose(out_phys, (2, 0, 1))


# speedup vs baseline: 1.4406x; 1.4406x over previous
import functools

import jax
import jax.numpy as jnp
from jax import lax
from jax.experimental import pallas as pl
from jax.experimental.pallas import tpu as pltpu
from jax.experimental.pallas import tpu_sc as plsc

VOCAB = 1000000
EMB = 64
NC = 2
NS = 16
NW = NC * NS
NPAIR = VOCAB // 2
NFULL = VOCAB // 128
TAIL_V0 = NFULL * 128
TAIL_W = 4

_MESH = plsc.VectorSubcoreMesh(
    core_axis_name="c", subcore_axis_name="s",
    num_cores=NC, num_subcores=NS,
)
_PARAMS = pltpu.CompilerParams(
    use_tc_tiling_on_sc=True, needs_layout_passes=False
)


def _wid():
    return lax.axis_index("s") * NC + lax.axis_index("c")


_TCV = 3584
_TCG = (VOCAB + _TCV - 1) // _TCV


def _relayout_body(x_ref, o_ref):
    y = x_ref[...].T
    o_ref[...] = jnp.concatenate([y, y], axis=1)


_phase_a = pl.pallas_call(
    _relayout_body,
    out_shape=jax.ShapeDtypeStruct((VOCAB, 128), jnp.float32),
    grid=(_TCG,),
    in_specs=[pl.BlockSpec((EMB, _TCV), lambda g: (0, g))],
    out_specs=pl.BlockSpec((_TCV, 128), lambda g: (g, 0)),
)


@functools.partial(
    pl.kernel,
    out_type=jax.ShapeDtypeStruct((200, EMB, 4096), jnp.float32),
    mesh=_MESH,
    scratch_types=[
        pltpu.VMEM((2, 128), jnp.int32),
        pltpu.VMEM((2, 128, 128), jnp.float32),
        pltpu.VMEM((2, EMB, 128), jnp.float32),
        pltpu.SemaphoreType.DMA((2,)),
        pltpu.SemaphoreType.DMA((2,)),
        pltpu.SemaphoreType.DMA((2,)),
    ],
    compiler_params=_PARAMS,
)
def _phase_b(tok_t, tabp, outp, idxb, rowb, outt, isem, gsem, wsem):
    w = _wid()
    iota = lax.iota(jnp.int32, 16)
    nunits = 200

    def unit_tb(i):
        u = w * nunits + i
        return u // 32, lax.rem(u, 32)

    def start_idx(i):
        t, bb = unit_tb(i)
        pltpu.async_copy(
            tok_t.at[t, pl.ds(bb * 128, 128)], idxb.at[i % 2], isem.at[i % 2]
        )

    start_idx(0)

    def step(i, _):
        buf = lax.rem(i, 2)
        pbuf = lax.rem(i + 1, 2)

        @pl.when(i + 1 < nunits)
        def _():
            start_idx(i + 1)

        pltpu.make_async_copy(
            tok_t.at[0, pl.ds(0, 128)], idxb.at[buf], isem.at[buf]
        ).wait()
        pltpu.async_copy(tabp.at[idxb.at[buf]], rowb.at[buf], gsem.at[buf])

        @pl.when(i >= 1)
        def _():
            t, bb = unit_tb(i - 1)
            pltpu.make_async_copy(
                tabp.at[pl.ds(0, 128)], rowb.at[pbuf], gsem.at[pbuf]
            ).wait()

            @pl.when(i >= 3)
            def _():
                pltpu.make_async_copy(
                    outt.at[pbuf], outp.at[0, :, pl.ds(0, 128)], wsem.at[pbuf]
                ).wait()

            @plsc.parallel_loop(0, EMB, 1, unroll=16)
            def _erow(e):
                cols = jnp.full((16,), e, jnp.int32)
                for k in range(8):
                    rows = iota + (16 * k)
                    outt[pbuf, e, pl.ds(16 * k, 16)] = plsc.load_gather(
                        rowb.at[pbuf], [rows, cols]
                    )
            pltpu.async_copy(
                outt.at[pbuf], outp.at[t, :, pl.ds(bb * 128, 128)],
                wsem.at[pbuf],
            )
        return ()

    lax.fori_loop(0, nunits, step, (), unroll=False)

    last = lax.rem(nunits - 1, 2)
    t, bb = unit_tb(nunits - 1)
    pltpu.make_async_copy(
        tabp.at[pl.ds(0, 128)], rowb.at[last], gsem.at[last]
    ).wait()
    pltpu.make_async_copy(
        outt.at[last], outp.at[0, :, pl.ds(0, 128)], wsem.at[last]
    ).wait()

    @plsc.parallel_loop(0, EMB, 1, unroll=16)
    def _erow(e):
        cols = jnp.full((16,), e, jnp.int32)
        for k in range(8):
            rows = iota + (16 * k)
            outt[last, e, pl.ds(16 * k, 16)] = plsc.load_gather(
                rowb.at[last], [rows, cols]
            )
    pltpu.sync_copy(outt.at[last], outp.at[t, :, pl.ds(bb * 128, 128)])
    pltpu.make_async_copy(
        outt.at[1 - last], outp.at[0, :, pl.ds(0, 128)], wsem.at[1 - last]
    ).wait()


def kernel(token_ids, embeddings):
    tok_t = token_ids.T
    emb_t = embeddings.T
    tabp = _phase_a(emb_t)
    out_phys = _phase_b(tok_t, tabp)
    return jnp.transpose(out_phys, (2, 0, 1))


# speedup vs baseline: 1.4407x; 1.0001x over previous
import functools

import jax
import jax.numpy as jnp
from jax import lax
from jax.experimental import pallas as pl
from jax.experimental.pallas import tpu as pltpu
from jax.experimental.pallas import tpu_sc as plsc

VOCAB = 1000000
EMB = 64
NC = 2
NS = 16
NW = NC * NS
NPAIR = VOCAB // 2
NFULL = VOCAB // 128
TAIL_V0 = NFULL * 128
TAIL_W = 4

_MESH = plsc.VectorSubcoreMesh(
    core_axis_name="c", subcore_axis_name="s",
    num_cores=NC, num_subcores=NS,
)
_PARAMS = pltpu.CompilerParams(
    use_tc_tiling_on_sc=True, needs_layout_passes=False
)


def _wid():
    return lax.axis_index("s") * NC + lax.axis_index("c")


_TCV = 3584
_TCG = TAIL_V0 // _TCV


def _relayout_body(x_ref, o_ref):
    y = x_ref[...].T
    o_ref[...] = jnp.concatenate([y, y], axis=1)


_phase_a = pl.pallas_call(
    _relayout_body,
    out_shape=jax.ShapeDtypeStruct((VOCAB, 128), jnp.float32),
    grid=(_TCG,),
    in_specs=[pl.BlockSpec((EMB, _TCV), lambda g: (0, g))],
    out_specs=pl.BlockSpec((_TCV, 128), lambda g: (g, 0)),
)


def _tail_body(tab_ref, tail_ref, o_ref):
    o_ref[...] = tail_ref[...]


_tail_fix = pl.pallas_call(
    _tail_body,
    out_shape=jax.ShapeDtypeStruct((VOCAB, 128), jnp.float32),
    grid=(1,),
    in_specs=[
        pl.BlockSpec((8, 128), lambda g: (0, 0)),
        pl.BlockSpec((64, 128), lambda g: (0, 0)),
    ],
    out_specs=pl.BlockSpec((64, 128), lambda g: (TAIL_V0 // 64, 0)),
    input_output_aliases={0: 0},
)


@functools.partial(
    pl.kernel,
    out_type=jax.ShapeDtypeStruct((200, EMB, 4096), jnp.float32),
    mesh=_MESH,
    scratch_types=[
        pltpu.VMEM((2, 128), jnp.int32),
        pltpu.VMEM((2, 128, 128), jnp.float32),
        pltpu.VMEM((2, EMB, 128), jnp.float32),
        pltpu.SemaphoreType.DMA((2,)),
        pltpu.SemaphoreType.DMA((2,)),
        pltpu.SemaphoreType.DMA((2,)),
    ],
    compiler_params=_PARAMS,
)
def _phase_b(tok_t, tabp, outp, idxb, rowb, outt, isem, gsem, wsem):
    w = _wid()
    iota = lax.iota(jnp.int32, 16)
    nunits = 200

    def unit_tb(i):
        u = w * nunits + i
        return u // 32, lax.rem(u, 32)

    def start_idx(i):
        t, bb = unit_tb(i)
        pltpu.async_copy(
            tok_t.at[t, pl.ds(bb * 128, 128)], idxb.at[i % 2], isem.at[i % 2]
        )

    start_idx(0)

    def step(i, _):
        buf = lax.rem(i, 2)
        pbuf = lax.rem(i + 1, 2)

        pltpu.make_async_copy(
            tok_t.at[0, pl.ds(0, 128)], idxb.at[buf], isem.at[buf]
        ).wait()
        pltpu.async_copy(tabp.at[idxb.at[buf]], rowb.at[buf], gsem.at[buf])

        @pl.when(i == 0)
        def _():
            start_idx(1)

        @pl.when(i >= 1)
        def _():
            t, bb = unit_tb(i - 1)
            pltpu.make_async_copy(
                tabp.at[pl.ds(0, 128)], rowb.at[pbuf], gsem.at[pbuf]
            ).wait()

            @pl.when(i + 1 < nunits)
            def _():
                start_idx(i + 1)

            @pl.when(i >= 3)
            def _():
                pltpu.make_async_copy(
                    outt.at[pbuf], outp.at[0, :, pl.ds(0, 128)], wsem.at[pbuf]
                ).wait()

            @plsc.parallel_loop(0, EMB, 1, unroll=16)
            def _erow(e):
                cols = jnp.full((16,), e, jnp.int32)
                for k in range(8):
                    rows = iota + (16 * k)
                    outt[pbuf, e, pl.ds(16 * k, 16)] = plsc.load_gather(
                        rowb.at[pbuf], [rows, cols]
                    )
            pltpu.async_copy(
                outt.at[pbuf], outp.at[t, :, pl.ds(bb * 128, 128)],
                wsem.at[pbuf],
            )
        return ()

    lax.fori_loop(0, nunits, step, (), unroll=False)

    last = lax.rem(nunits - 1, 2)
    t, bb = unit_tb(nunits - 1)
    pltpu.make_async_copy(
        tabp.at[pl.ds(0, 128)], rowb.at[last], gsem.at[last]
    ).wait()
    pltpu.make_async_copy(
        outt.at[last], outp.at[0, :, pl.ds(0, 128)], wsem.at[last]
    ).wait()

    @plsc.parallel_loop(0, EMB, 1, unroll=16)
    def _erow(e):
        cols = jnp.full((16,), e, jnp.int32)
        for k in range(8):
            rows = iota + (16 * k)
            outt[last, e, pl.ds(16 * k, 16)] = plsc.load_gather(
                rowb.at[last], [rows, cols]
            )
    pltpu.sync_copy(outt.at[last], outp.at[t, :, pl.ds(bb * 128, 128)])
    pltpu.make_async_copy(
        outt.at[1 - last], outp.at[0, :, pl.ds(0, 128)], wsem.at[1 - last]
    ).wait()


def kernel(token_ids, embeddings):
    tok_t = token_ids.T
    emb_t = embeddings.T
    tabp = _phase_a(emb_t)
    tail3 = embeddings[TAIL_V0:]
    tabp = _tail_fix(tabp, jnp.concatenate([tail3, tail3], axis=1))
    out_phys = _phase_b(tok_t, tabp)
    return jnp.transpose(out_phys, (2, 0, 1))
